# baseline (speedup 1.0000x reference)
import jax
import jax.numpy as jnp
from jax.experimental import pallas as pl
from jax.experimental.pallas import tpu as pltpu

_N = 10000
_D = 128
_TILE = 10000


def _fused_gru_kernel(x_ref, wz_ref, wh_ref, bz_ref, bh_ref, wl_ref, bl_ref,
                      out_ref):
    wz = wz_ref[0, 0, :_D, :] + wz_ref[1, 0, :_D, :]
    wh = wh_ref[0, 0, :_D, :] + wh_ref[1, 0, :_D, :]
    w_cat = jnp.concatenate([wz, wh], axis=1)
    xb = x_ref[...]
    a = jnp.dot(xb, w_cat, preferred_element_type=jnp.float32)
    az = a[:, :_D] + bz_ref[...]
    ah = a[:, _D:] + bh_ref[...]
    z = 0.5 + 0.5 * jnp.tanh(0.5 * az)
    ht = jnp.tanh(ah)
    h = jnp.maximum((1.0 - z) * ht, 0.0)
    out_ref[...] = (
        jnp.dot(h, wl_ref[...], preferred_element_type=jnp.float32)
        + bl_ref[...])


def kernel(x, edge_index, edge_weight, Wz, bz, Wr, br, Wh, bh, W_lin, b_lin):
    del edge_index, edge_weight, Wr, br
    bz2 = bz.reshape(1, _D)
    bh2 = bh.reshape(1, _D)
    wl = W_lin.T
    bl2 = b_lin.reshape(1, 1)

    out = pl.pallas_call(
        _fused_gru_kernel,
        grid=(_N // _TILE,),
        in_specs=[
            pl.BlockSpec((_TILE, _D), lambda i: (i, 0)),
            pl.BlockSpec((2, 1, 2 * _D, _D), lambda i: (0, 0, 0, 0)),
            pl.BlockSpec((2, 1, 2 * _D, _D), lambda i: (0, 0, 0, 0)),
            pl.BlockSpec((1, _D), lambda i: (0, 0)),
            pl.BlockSpec((1, _D), lambda i: (0, 0)),
            pl.BlockSpec((_D, 1), lambda i: (0, 0)),
            pl.BlockSpec((1, 1), lambda i: (0, 0)),
        ],
        out_specs=pl.BlockSpec((_TILE, 1), lambda i: (i, 0)),
        out_shape=jax.ShapeDtypeStruct((_N, 1), jnp.float32),
        compiler_params=pltpu.CompilerParams(
            dimension_semantics=("arbitrary",)),
    )(x, Wz, Wh, bz2, bh2, wl, bl2)
    return out


# speedup vs baseline: 1.0136x; 1.0136x over previous
import jax
import jax.numpy as jnp
from jax.experimental import pallas as pl
from jax.experimental.pallas import tpu as pltpu

_N = 10000
_D = 128
_TILE = 3336


def _fused_gru_kernel(x_ref, wz_ref, wh_ref, bz_ref, bh_ref, wl_ref, bl_ref,
                      out_ref):
    wz = wz_ref[0, 0, :_D, :] + wz_ref[1, 0, :_D, :]
    wh = wh_ref[0, 0, :_D, :] + wh_ref[1, 0, :_D, :]
    w_cat = jnp.concatenate([wz, wh], axis=1)
    xb = x_ref[...]
    a = jnp.dot(xb, w_cat, preferred_element_type=jnp.float32)
    az = a[:, :_D] + bz_ref[...]
    ah = a[:, _D:] + bh_ref[...]
    z = 0.5 + 0.5 * jnp.tanh(0.5 * az)
    ht = jnp.tanh(ah)
    h = jnp.maximum((1.0 - z) * ht, 0.0)
    out_ref[...] = (
        jnp.dot(h, wl_ref[...], preferred_element_type=jnp.float32)
        + bl_ref[...])


def kernel(x, edge_index, edge_weight, Wz, bz, Wr, br, Wh, bh, W_lin, b_lin):
    del edge_index, edge_weight, Wr, br
    bz2 = bz.reshape(1, _D)
    bh2 = bh.reshape(1, _D)
    wl = W_lin.T
    bl2 = b_lin.reshape(1, 1)

    out = pl.pallas_call(
        _fused_gru_kernel,
        grid=(pl.cdiv(_N, _TILE),),
        in_specs=[
            pl.BlockSpec((_TILE, _D), lambda i: (i, 0)),
            pl.BlockSpec((2, 1, 2 * _D, _D), lambda i: (0, 0, 0, 0)),
            pl.BlockSpec((2, 1, 2 * _D, _D), lambda i: (0, 0, 0, 0)),
            pl.BlockSpec((1, _D), lambda i: (0, 0)),
            pl.BlockSpec((1, _D), lambda i: (0, 0)),
            pl.BlockSpec((_D, 1), lambda i: (0, 0)),
            pl.BlockSpec((1, 1), lambda i: (0, 0)),
        ],
        out_specs=pl.BlockSpec((_TILE, 1), lambda i: (i, 0)),
        out_shape=jax.ShapeDtypeStruct((_N, 1), jnp.float32),
        compiler_params=pltpu.CompilerParams(
            dimension_semantics=("arbitrary",)),
    )(x, Wz, Wh, bz2, bh2, wl, bl2)
    return out


# speedup vs baseline: 1.0716x; 1.0573x over previous
import jax
import jax.numpy as jnp
from jax.experimental import pallas as pl
from jax.experimental.pallas import tpu as pltpu

_N = 10000
_D = 128
_TILE = 5000


def _fused_gru_kernel(x_ref, wz_ref, wh_ref, bz_ref, bh_ref, wl_ref, bl_ref,
                      out_ref):
    wz = 0.5 * (wz_ref[0, 0, :_D, :] + wz_ref[1, 0, :_D, :])
    wh = wh_ref[0, 0, :_D, :] + wh_ref[1, 0, :_D, :]
    w_cat = jnp.concatenate([wz, wh], axis=1)
    xb = x_ref[...]
    a = jnp.dot(xb, w_cat, preferred_element_type=jnp.float32)
    t = jnp.tanh(a[:, :_D] + 0.5 * bz_ref[...])
    ht = jnp.tanh(a[:, _D:] + bh_ref[...])
    h = jnp.maximum((1.0 - t) * ht, 0.0)
    out_ref[...] = (
        jnp.dot(h, 0.5 * wl_ref[...], preferred_element_type=jnp.float32)
        + bl_ref[...])


def kernel(x, edge_index, edge_weight, Wz, bz, Wr, br, Wh, bh, W_lin, b_lin):
    del edge_index, edge_weight, Wr, br
    bz2 = bz.reshape(1, _D)
    bh2 = bh.reshape(1, _D)
    wl = W_lin.T
    bl2 = b_lin.reshape(1, 1)

    out = pl.pallas_call(
        _fused_gru_kernel,
        grid=(pl.cdiv(_N, _TILE),),
        in_specs=[
            pl.BlockSpec((_TILE, _D), lambda i: (i, 0)),
            pl.BlockSpec((2, 1, 2 * _D, _D), lambda i: (0, 0, 0, 0)),
            pl.BlockSpec((2, 1, 2 * _D, _D), lambda i: (0, 0, 0, 0)),
            pl.BlockSpec((1, _D), lambda i: (0, 0)),
            pl.BlockSpec((1, _D), lambda i: (0, 0)),
            pl.BlockSpec((_D, 1), lambda i: (0, 0)),
            pl.BlockSpec((1, 1), lambda i: (0, 0)),
        ],
        out_specs=pl.BlockSpec((_TILE, 1), lambda i: (i, 0)),
        out_shape=jax.ShapeDtypeStruct((_N, 1), jnp.float32),
        compiler_params=pltpu.CompilerParams(
            dimension_semantics=("arbitrary",)),
    )(x, Wz, Wh, bz2, bh2, wl, bl2)
    return out
